# baseline (speedup 1.0000x reference)
import functools

import jax
import jax.numpy as jnp
from jax import lax
from jax.experimental import pallas as pl
from jax.experimental.pallas import tpu as pltpu
from jax.experimental.pallas import tpu_sc as plsc

_INPUT_SIZE = 26
_BATCH = 16384
_DIM = 32
_TABLE = 1 << 20
_MASK = _TABLE - 1
_NW = 32
_BPW = _BATCH // _NW
_L = 16
_NCHUNK = _BPW // _L
_GATHER = 128
_NGATHER = _BPW // _GATHER
_LINES = _TABLE // 4
_RMAGIC = 12582912.0

_mesh = plsc.VectorSubcoreMesh(core_axis_name="c", subcore_axis_name="s")


@functools.partial(
    pl.kernel,
    mesh=_mesh,
    out_type=jax.ShapeDtypeStruct((_BATCH, _DIM), jnp.float32),
    scratch_types=[
        pltpu.VMEM((_INPUT_SIZE, _BPW), jnp.float32),
        pltpu.VMEM((8, 128), jnp.int32),
        pltpu.VMEM((_NGATHER, _GATHER), jnp.int32),
        pltpu.VMEM((_BPW,), jnp.int32),
        pltpu.VMEM((2, _GATHER, 128), jnp.float32),
        pltpu.VMEM((_BPW, _DIM), jnp.float32),
        pltpu.SemaphoreType.DMA,
    ],
)
def _hash_embed(x_hbm, hb_hbm, feat_hbm, out_hbm,
                xt_v, hb_v, idx_v, sub_v, lines_v, rows_v, sem):
    wid = lax.axis_index("s") * 2 + lax.axis_index("c")
    base = wid * _BPW

    pltpu.sync_copy(x_hbm.at[wid], xt_v)
    pltpu.sync_copy(hb_hbm, hb_v)

    def _coef(level, i):
        return hb_v[4 * level + i // 8, pl.ds((i % 8) * _L, _L)]

    dsplat = [_coef(1, i) - _coef(0, i) for i in range(_INPUT_SIZE)]
    h0vec = _coef(0, 0)
    for i in range(1, _INPUT_SIZE):
        h0vec = h0vec + _coef(0, i)
    zero = jnp.zeros((_L,), jnp.int32)

    for c in range(_NCHUNK):
        acc = h0vec
        for i in range(_INPUT_SIZE):
            xv = xt_v[i, pl.ds(c * _L, _L)]
            acc = acc + jnp.where(xv > 0.5, dsplat[i], zero)
        idx = jnp.bitwise_and(acc, _MASK)
        g, off = divmod(c * _L, _GATHER)
        idx_v[g, pl.ds(off, _L)] = jnp.right_shift(idx, 2)
        sub_v[pl.ds(c * _L, _L)] = jnp.bitwise_and(idx, 3)

    def _start(g):
        return pltpu.async_copy(feat_hbm.at[idx_v.at[g]],
                                lines_v.at[g % 2], sem)

    cp = _start(0)
    for g in range(_NGATHER):
        nxt = _start(g + 1) if g + 1 < _NGATHER else None
        cp.wait()

        def _extract(c, carry, g=g):
            sv = sub_v[pl.ds(g * _GATHER + c * _L, _L)] * _DIM
            for r in range(_L):
                off = sv[r]
                b = c * _L + r
                for h in range(_DIM // _L):
                    v = lines_v[g % 2, b, pl.ds(off + h * _L, _L)]
                    v = jnp.minimum(jnp.maximum(v, -1.0), 127.0 / 128.0)
                    y = v * 128.0
                    q = (y + _RMAGIC) - _RMAGIC
                    rows_v[g * _GATHER + b, pl.ds(h * _L, _L)] = q * (1.0 / 128.0)
            return carry

        lax.fori_loop(0, _GATHER // _L, _extract, 0)
        cp = nxt

    pltpu.sync_copy(rows_v, out_hbm.at[pl.ds(base, _BPW)])


def kernel(x, features, hashs):
    xt = x.T.reshape(_INPUT_SIZE, _NW, _BPW).transpose(1, 0, 2)
    feat_lines = features.reshape(_LINES, 128)
    hpad = jnp.zeros((2, 32), jnp.int32).at[:, :_INPUT_SIZE].set(hashs.T)
    hb = jnp.repeat(hpad, _L, axis=1).reshape(8, 128)
    return _hash_embed(xt, hb, feat_lines)


# speedup vs baseline: 1.0130x; 1.0130x over previous
import functools

import jax
import jax.numpy as jnp
from jax import lax
from jax.experimental import pallas as pl
from jax.experimental.pallas import tpu as pltpu
from jax.experimental.pallas import tpu_sc as plsc

_INPUT_SIZE = 26
_BATCH = 16384
_DIM = 32
_TABLE = 1 << 20
_MASK = _TABLE - 1
_NW = 32
_BPW = _BATCH // _NW
_L = 16
_NCHUNK = _BPW // _L
_GATHER = 128
_NGATHER = _BPW // _GATHER
_RMAGIC = 12582912.0

_mesh = plsc.VectorSubcoreMesh(core_axis_name="c", subcore_axis_name="s")


@functools.partial(
    pl.kernel,
    mesh=_mesh,
    out_type=jax.ShapeDtypeStruct((_BATCH, _DIM), jnp.float32),
    scratch_types=[
        pltpu.VMEM((_INPUT_SIZE, _BPW), jnp.float32),
        pltpu.VMEM((8, 128), jnp.int32),
        pltpu.VMEM((_NGATHER, _GATHER), jnp.int32),
        pltpu.VMEM((2, _GATHER, _DIM), jnp.float32),
        pltpu.VMEM((_BPW, _DIM), jnp.float32),
        pltpu.SemaphoreType.DMA,
    ],
    compiler_params=pltpu.CompilerParams(use_tc_tiling_on_sc=False),
)
def _hash_embed(x_hbm, hb_hbm, feat_hbm, out_hbm,
                xt_v, hb_v, idx_v, gath_v, rows_v, sem):
    wid = lax.axis_index("s") * 2 + lax.axis_index("c")
    base = wid * _BPW

    pltpu.sync_copy(x_hbm.at[wid], xt_v)
    pltpu.sync_copy(hb_hbm, hb_v)

    def _coef(level, i):
        return hb_v[4 * level + i // 8, pl.ds((i % 8) * _L, _L)]

    dsplat = [_coef(1, i) - _coef(0, i) for i in range(_INPUT_SIZE)]
    h0vec = _coef(0, 0)
    for i in range(1, _INPUT_SIZE):
        h0vec = h0vec + _coef(0, i)
    zero = jnp.zeros((_L,), jnp.int32)

    for c in range(_NCHUNK):
        acc = h0vec
        for i in range(_INPUT_SIZE):
            xv = xt_v[i, pl.ds(c * _L, _L)]
            acc = acc + jnp.where(xv > 0.5, dsplat[i], zero)
        idx = jnp.bitwise_and(acc, _MASK)
        g, off = divmod(c * _L, _GATHER)
        idx_v[g, pl.ds(off, _L)] = idx

    def _start(g):
        return pltpu.async_copy(feat_hbm.at[idx_v.at[g]],
                                gath_v.at[g % 2], sem)

    cp = _start(0)
    for g in range(_NGATHER):
        nxt = _start(g + 1) if g + 1 < _NGATHER else None
        cp.wait()

        def _quant(b, carry, g=g):
            for h in range(_DIM // _L):
                v = gath_v[g % 2, b, pl.ds(h * _L, _L)]
                v = jnp.minimum(jnp.maximum(v, -1.0), 127.0 / 128.0)
                y = v * 128.0
                q = (y + _RMAGIC) - _RMAGIC
                rows_v[g * _GATHER + b, pl.ds(h * _L, _L)] = q * (1.0 / 128.0)
            return carry

        lax.fori_loop(0, _GATHER, _quant, 0)
        cp = nxt

    pltpu.sync_copy(rows_v, out_hbm.at[pl.ds(base, _BPW)])


def kernel(x, features, hashs):
    xt = x.T.reshape(_INPUT_SIZE, _NW, _BPW).transpose(1, 0, 2)
    hpad = jnp.zeros((2, 32), jnp.int32).at[:, :_INPUT_SIZE].set(hashs.T)
    hb = jnp.repeat(hpad, _L, axis=1).reshape(8, 128)
    return _hash_embed(xt, hb, features)


# speedup vs baseline: 1.0159x; 1.0029x over previous
import functools

import jax
import jax.numpy as jnp
from jax import lax
from jax.experimental import pallas as pl
from jax.experimental.pallas import tpu as pltpu
from jax.experimental.pallas import tpu_sc as plsc

_INPUT_SIZE = 26
_BATCH = 16384
_DIM = 32
_TABLE = 1 << 20
_MASK = _TABLE - 1
_NW = 32
_BPW = _BATCH // _NW
_L = 16
_GATHER = 128
_NGATHER = _BPW // _GATHER
_TCB = 2048
_RMAGIC = 12582912.0

_mesh = plsc.VectorSubcoreMesh(core_axis_name="c", subcore_axis_name="s")


def _hash_tc(x_ref, d_ref, h0_ref, out_ref):
    bits = x_ref[...] > 0.5
    contrib = jnp.where(bits, d_ref[...], 0)
    h = h0_ref[0, 0] + jnp.sum(contrib, axis=1, dtype=jnp.int32)
    out_ref[...] = jnp.bitwise_and(h, _MASK).reshape(_TCB // 128, 128)


_hash_idx = pl.pallas_call(
    _hash_tc,
    grid=(_BATCH // _TCB,),
    in_specs=[
        pl.BlockSpec((_TCB, _INPUT_SIZE), lambda i: (i, 0)),
        pl.BlockSpec((1, _INPUT_SIZE), lambda i: (0, 0)),
        pl.BlockSpec((1, 1), lambda i: (0, 0)),
    ],
    out_specs=pl.BlockSpec((_TCB // 128, 128), lambda i: (i, 0)),
    out_shape=jax.ShapeDtypeStruct((_BATCH // 128, 128), jnp.int32),
)


@functools.partial(
    pl.kernel,
    mesh=_mesh,
    out_type=jax.ShapeDtypeStruct((_BATCH, _DIM), jnp.float32),
    scratch_types=[
        pltpu.VMEM((_NGATHER, _GATHER), jnp.int32),
        pltpu.VMEM((2, _GATHER, _DIM), jnp.float32),
        pltpu.VMEM((_BPW, _DIM), jnp.float32),
        pltpu.SemaphoreType.DMA,
    ],
    compiler_params=pltpu.CompilerParams(use_tc_tiling_on_sc=False),
)
def _gather_quant(idx_hbm, feat_hbm, out_hbm, idx_v, gath_v, rows_v, sem):
    wid = lax.axis_index("s") * 2 + lax.axis_index("c")
    base = wid * _BPW

    pltpu.sync_copy(idx_hbm.at[pl.ds(wid * _NGATHER, _NGATHER)], idx_v)

    def _start(g):
        return pltpu.async_copy(feat_hbm.at[idx_v.at[g]],
                                gath_v.at[g % 2], sem)

    cp = _start(0)
    for g in range(_NGATHER):
        nxt = _start(g + 1) if g + 1 < _NGATHER else None
        cp.wait()

        def _quant(b, carry, g=g):
            for h in range(_DIM // _L):
                v = gath_v[g % 2, b, pl.ds(h * _L, _L)]
                v = jnp.minimum(jnp.maximum(v, -1.0), 127.0 / 128.0)
                y = v * 128.0
                q = (y + _RMAGIC) - _RMAGIC
                rows_v[g * _GATHER + b, pl.ds(h * _L, _L)] = q * (1.0 / 128.0)
            return carry

        lax.fori_loop(0, _GATHER, _quant, 0)
        cp = nxt

    pltpu.sync_copy(rows_v, out_hbm.at[pl.ds(base, _BPW)])


def kernel(x, features, hashs):
    hi = hashs.astype(jnp.int32)
    dv = (hi[:, 1] - hi[:, 0]).reshape(1, _INPUT_SIZE)
    h0 = jnp.sum(hi[:, 0], dtype=jnp.int32).reshape(1, 1)
    idx = _hash_idx(x, dv, h0)
    return _gather_quant(idx, features)
